# Initial kernel scaffold; baseline (speedup 1.0000x reference)
import jax
import jax.numpy as jnp
from jax.experimental import pallas as pl


def kernel(log_probs):
    raise NotImplementedError("write your pallas kernel here")


# speedup vs baseline: 8.5999x; 8.5999x over previous
import jax
import jax.numpy as jnp
from jax.experimental import pallas as pl
from jax.experimental.pallas import tpu as pltpu

T = 8
B = 32
K = 4
V = 32000
EOS = 2
R = B * K
NROWS = T * R

RB = 128
VC = 3200
NRB = NROWS // RB
NVC = V // VC

_NEG_INF = float("-inf")
_BIG_I32 = 2 ** 30


def _top4_scan_kernel(x_ref, vals_ref, vidx_ref):
    j = pl.program_id(1)
    x = x_ref[...]
    iota = jax.lax.broadcasted_iota(jnp.int32, (RB, VC), 1) + j * VC

    cv = []
    ci = []
    for _ in range(4):
        m = jnp.max(x, axis=1, keepdims=True)
        eq = x == m
        am = jnp.min(jnp.where(eq, iota, _BIG_I32), axis=1, keepdims=True)
        cv.append(m)
        ci.append(am)
        x = jnp.where(iota == am, _NEG_INF, x)
    chunk_v = jnp.concatenate(cv, axis=1)
    chunk_i = jnp.concatenate(ci, axis=1)

    @pl.when(j == 0)
    def _init():
        vals_ref[...] = chunk_v
        vidx_ref[...] = chunk_i

    @pl.when(j > 0)
    def _merge():
        av = jnp.concatenate([vals_ref[...], chunk_v], axis=1)
        ai = jnp.concatenate([vidx_ref[...], chunk_i], axis=1)
        ov = []
        oi = []
        x8 = av
        for _ in range(4):
            m = jnp.max(x8, axis=1, keepdims=True)
            eq = x8 == m
            am = jnp.min(jnp.where(eq, ai, _BIG_I32), axis=1, keepdims=True)
            ov.append(m)
            oi.append(am)
            x8 = jnp.where(eq & (ai == am), _NEG_INF, x8)
        vals_ref[...] = jnp.concatenate(ov, axis=1)
        vidx_ref[...] = jnp.concatenate(oi, axis=1)


def _beam_kernel(tv_ref, ti_ref, seq_ref, ss_ref):
    lane16 = jax.lax.broadcasted_iota(jnp.int32, (B, 16), 1)
    kk = lane16 // 4
    lane4 = jax.lax.broadcasted_iota(jnp.int32, (B, K), 1)
    neg = jnp.float32(_NEG_INF)

    s = jnp.where(lane4 == 0, jnp.float32(0.0), neg)

    sym_hist = []
    pred_hist = []
    last_scores = None
    for t in range(T):
        tv = tv_ref[t]
        ti = ti_ref[t]
        s16 = jnp.where(kk == 0, s[:, 0:1],
              jnp.where(kk == 1, s[:, 1:2],
              jnp.where(kk == 2, s[:, 2:3], s[:, 3:4])))
        cand = s16 + tv
        flat = kk * V + ti

        sv = []
        sf = []
        x = cand
        for _ in range(4):
            m = jnp.max(x, axis=1, keepdims=True)
            eq = x == m
            am = jnp.min(jnp.where(eq, flat, _BIG_I32), axis=1, keepdims=True)
            sv.append(m)
            sf.append(am)
            x = jnp.where(eq & (flat == am), neg, x)
        sc = jnp.concatenate(sv, axis=1)
        fl = jnp.concatenate(sf, axis=1)
        sym = jnp.remainder(fl, V)
        pk = fl // V
        sym_hist.append(sym)
        pred_hist.append(pk)
        last_scores = sc
        s = jnp.where(sym == EOS, neg, sc)

    pv = []
    pi = []
    x = last_scores
    for _ in range(4):
        m = jnp.max(x, axis=1, keepdims=True)
        eq = x == m
        am = jnp.min(jnp.where(eq, lane4, _BIG_I32), axis=1, keepdims=True)
        pv.append(m)
        pi.append(am)
        x = jnp.where(eq & (lane4 == am), neg, x)
    ss_ref[...] = jnp.concatenate(pv, axis=1)
    tp = jnp.concatenate(pi, axis=1)

    def gather4(val, idx):
        acc = jnp.broadcast_to(val[:, 0:1], (B, K))
        for kslot in range(1, K):
            acc = jnp.where(idx == kslot, val[:, kslot:kslot + 1], acc)
        return acc

    for t in range(T - 1, -1, -1):
        seq_ref[t] = gather4(sym_hist[t], tp)
        tp = gather4(pred_hist[t], tp)


@jax.jit
def kernel(log_probs):
    lp = log_probs.reshape(NROWS, V)

    vals, vidx = pl.pallas_call(
        _top4_scan_kernel,
        grid=(NRB, NVC),
        in_specs=[pl.BlockSpec((RB, VC), lambda i, j: (i, j))],
        out_specs=[
            pl.BlockSpec((RB, 4), lambda i, j: (i, 0)),
            pl.BlockSpec((RB, 4), lambda i, j: (i, 0)),
        ],
        out_shape=[
            jax.ShapeDtypeStruct((NROWS, 4), jnp.float32),
            jax.ShapeDtypeStruct((NROWS, 4), jnp.int32),
        ],
    )(lp)

    tv = vals.reshape(T, B, K * 4)
    ti = vidx.reshape(T, B, K * 4)

    sequences, sorted_scores = pl.pallas_call(
        _beam_kernel,
        in_specs=[
            pl.BlockSpec((T, B, K * 4), lambda: (0, 0, 0)),
            pl.BlockSpec((T, B, K * 4), lambda: (0, 0, 0)),
        ],
        out_specs=[
            pl.BlockSpec((T, B, K), lambda: (0, 0, 0)),
            pl.BlockSpec((B, K), lambda: (0, 0)),
        ],
        out_shape=[
            jax.ShapeDtypeStruct((T, B, K), jnp.int32),
            jax.ShapeDtypeStruct((B, K), jnp.float32),
        ],
    )(tv, ti)

    return sequences, sorted_scores


# speedup vs baseline: 14.2459x; 1.6565x over previous
import jax
import jax.numpy as jnp
from jax.experimental import pallas as pl
from jax.experimental.pallas import tpu as pltpu

T = 8
B = 32
K = 4
V = 32000
EOS = 2
R = B * K
NROWS = T * R

RB = 128
VC = 32000
NRB = NROWS // RB
NVC = V // VC

_NEG_INF = float("-inf")
_BIG_I32 = 2 ** 30


def _top4_scan_kernel(x_ref, vals_ref, vidx_ref):
    x = x_ref[...]
    iota = jax.lax.broadcasted_iota(jnp.int32, (RB, VC), 1)
    if NVC > 1:
        iota = iota + pl.program_id(1) * VC

    cv = []
    ci = []
    for p in range(4):
        m = jnp.max(x, axis=1, keepdims=True)
        eq = x == m
        am = jnp.min(jnp.where(eq, iota, _BIG_I32), axis=1, keepdims=True)
        cv.append(m)
        ci.append(am)
        if p < 3:
            x = jnp.where(iota == am, _NEG_INF, x)
    chunk_v = jnp.concatenate(cv, axis=1)
    chunk_i = jnp.concatenate(ci, axis=1)

    if NVC == 1:
        vals_ref[...] = chunk_v
        vidx_ref[...] = chunk_i
    else:
        j = pl.program_id(1)

        @pl.when(j == 0)
        def _init():
            vals_ref[...] = chunk_v
            vidx_ref[...] = chunk_i

        @pl.when(j > 0)
        def _merge():
            av = jnp.concatenate([vals_ref[...], chunk_v], axis=1)
            ai = jnp.concatenate([vidx_ref[...], chunk_i], axis=1)
            ov = []
            oi = []
            x8 = av
            for _ in range(4):
                m = jnp.max(x8, axis=1, keepdims=True)
                eq = x8 == m
                am = jnp.min(jnp.where(eq, ai, _BIG_I32), axis=1, keepdims=True)
                ov.append(m)
                oi.append(am)
                x8 = jnp.where(eq & (ai == am), _NEG_INF, x8)
            vals_ref[...] = jnp.concatenate(ov, axis=1)
            vidx_ref[...] = jnp.concatenate(oi, axis=1)


def _beam_kernel(tv_ref, ti_ref, seq_ref, ss_ref):
    lane16 = jax.lax.broadcasted_iota(jnp.int32, (B, 16), 1)
    kk = lane16 // 4
    lane4 = jax.lax.broadcasted_iota(jnp.int32, (B, K), 1)
    neg = jnp.float32(_NEG_INF)

    s = jnp.where(lane4 == 0, jnp.float32(0.0), neg)

    sym_hist = []
    pred_hist = []
    last_scores = None
    for t in range(T):
        tv = tv_ref[t]
        ti = ti_ref[t]
        s16 = jnp.where(kk == 0, s[:, 0:1],
              jnp.where(kk == 1, s[:, 1:2],
              jnp.where(kk == 2, s[:, 2:3], s[:, 3:4])))
        cand = s16 + tv
        flat = kk * V + ti

        sv = []
        sf = []
        x = cand
        for _ in range(4):
            m = jnp.max(x, axis=1, keepdims=True)
            eq = x == m
            am = jnp.min(jnp.where(eq, flat, _BIG_I32), axis=1, keepdims=True)
            sv.append(m)
            sf.append(am)
            x = jnp.where(eq & (flat == am), neg, x)
        sc = jnp.concatenate(sv, axis=1)
        fl = jnp.concatenate(sf, axis=1)
        sym = jnp.remainder(fl, V)
        pk = fl // V
        sym_hist.append(sym)
        pred_hist.append(pk)
        last_scores = sc
        s = jnp.where(sym == EOS, neg, sc)

    pv = []
    pi = []
    x = last_scores
    for _ in range(4):
        m = jnp.max(x, axis=1, keepdims=True)
        eq = x == m
        am = jnp.min(jnp.where(eq, lane4, _BIG_I32), axis=1, keepdims=True)
        pv.append(m)
        pi.append(am)
        x = jnp.where(eq & (lane4 == am), neg, x)
    ss_ref[...] = jnp.concatenate(pv, axis=1)
    tp = jnp.concatenate(pi, axis=1)

    def gather4(val, idx):
        acc = jnp.broadcast_to(val[:, 0:1], (B, K))
        for kslot in range(1, K):
            acc = jnp.where(idx == kslot, val[:, kslot:kslot + 1], acc)
        return acc

    for t in range(T - 1, -1, -1):
        seq_ref[t] = gather4(sym_hist[t], tp)
        tp = gather4(pred_hist[t], tp)


@jax.jit
def kernel(log_probs):
    lp = log_probs.reshape(NROWS, V)

    vals, vidx = pl.pallas_call(
        _top4_scan_kernel,
        grid=(NRB, NVC),
        in_specs=[pl.BlockSpec((RB, VC), lambda i, j: (i, j))],
        out_specs=[
            pl.BlockSpec((RB, 4), lambda i, j: (i, 0)),
            pl.BlockSpec((RB, 4), lambda i, j: (i, 0)),
        ],
        out_shape=[
            jax.ShapeDtypeStruct((NROWS, 4), jnp.float32),
            jax.ShapeDtypeStruct((NROWS, 4), jnp.int32),
        ],
    )(lp)

    tv = vals.reshape(T, B, K * 4)
    ti = vidx.reshape(T, B, K * 4)

    sequences, sorted_scores = pl.pallas_call(
        _beam_kernel,
        in_specs=[
            pl.BlockSpec((T, B, K * 4), lambda: (0, 0, 0)),
            pl.BlockSpec((T, B, K * 4), lambda: (0, 0, 0)),
        ],
        out_specs=[
            pl.BlockSpec((T, B, K), lambda: (0, 0, 0)),
            pl.BlockSpec((B, K), lambda: (0, 0)),
        ],
        out_shape=[
            jax.ShapeDtypeStruct((T, B, K), jnp.int32),
            jax.ShapeDtypeStruct((B, K), jnp.float32),
        ],
    )(tv, ti)

    return sequences, sorted_scores
